# Initial kernel scaffold; baseline (speedup 1.0000x reference)
import jax
import jax.numpy as jnp
from jax.experimental import pallas as pl


def kernel(x, edge_index, W1, W2):
    raise NotImplementedError("write your pallas kernel here")


# speedup vs baseline: 34.1722x; 34.1722x over previous
import functools

import jax
import jax.numpy as jnp
from jax import lax
from jax.experimental import pallas as pl
from jax.experimental.pallas import tpu as pltpu
from jax.experimental.pallas import tpu_sc as plsc

N = 10000
D = 128
F = 16
E = 320000
NC, NS = 2, 16
NW = NC * NS
BK = 128
NB = 79
EPT = NB * BK
EPAD = NW * EPT
NPAD = 10240
RPT = NPAD // NS

_mesh = plsc.VectorSubcoreMesh(core_axis_name="c", subcore_axis_name="s")
_sc_params = pltpu.CompilerParams(use_tc_tiling_on_sc=False)


@functools.partial(
    pl.kernel,
    out_type=jax.ShapeDtypeStruct((NC, NPAD), jnp.float32),
    mesh=_mesh,
    compiler_params=_sc_params,
    scratch_types=[
        pltpu.VMEM_SHARED((NPAD,), jnp.float32),
        pltpu.VMEM((NB, BK), jnp.int32),
        pltpu.VMEM((BK,), jnp.float32),
        pltpu.VMEM((RPT,), jnp.float32),
    ],
)
def _sc_degree(dstg_hbm, out_hbm, accum, dst_v, ones_v, buf_v):
    c = lax.axis_index("c")
    s = lax.axis_index("s")
    tid = c * NS + s
    z16 = jnp.zeros((16,), jnp.float32)
    for j in range(RPT // 16):
        buf_v[pl.ds(j * 16, 16)] = z16
    o16 = jnp.full((16,), 1.0, jnp.float32)
    for j in range(BK // 16):
        ones_v[pl.ds(j * 16, 16)] = o16
    pltpu.sync_copy(buf_v, accum.at[pl.ds(s * RPT, RPT)])
    pltpu.sync_copy(dstg_hbm.at[tid], dst_v)
    plsc.subcore_barrier()

    def step(b, carry):
        pltpu.sync_copy(ones_v, accum.at[dst_v.at[b]], add=True)
        return carry

    lax.fori_loop(0, NB, step, 0)
    plsc.subcore_barrier()
    pltpu.sync_copy(accum.at[pl.ds(s * RPT, RPT)], buf_v)
    pltpu.sync_copy(buf_v, out_hbm.at[c, pl.ds(s * RPT, RPT)])


@functools.partial(
    pl.kernel,
    out_type=jax.ShapeDtypeStruct((NC, NPAD, F), jnp.float32),
    mesh=_mesh,
    compiler_params=_sc_params,
    scratch_types=[
        pltpu.VMEM_SHARED((NPAD, F), jnp.float32),
        pltpu.VMEM((RPT, F), jnp.float32),
        pltpu.VMEM((BK, F), jnp.float32),
        pltpu.VMEM((NB, BK), jnp.int32),
        pltpu.VMEM((NB, BK), jnp.int32),
        pltpu.SemaphoreType.DMA,
    ],
)
def _sc_aggregate(u_hbm, srcg_hbm, dstg_hbm, out_hbm,
                  accum, row_buf, rows_v, src_v, dst_v, sem):
    c = lax.axis_index("c")
    s = lax.axis_index("s")
    tid = c * NS + s
    pltpu.sync_copy(u_hbm.at[pl.ds(s * RPT, RPT)], row_buf)
    pltpu.sync_copy(row_buf, accum.at[pl.ds(s * RPT, RPT)])
    pltpu.sync_copy(srcg_hbm.at[tid], src_v)
    pltpu.sync_copy(dstg_hbm.at[tid], dst_v)
    plsc.subcore_barrier()

    def step(b, carry):
        pltpu.async_copy(u_hbm.at[src_v.at[b]], rows_v, sem).wait()
        pltpu.sync_copy(rows_v, accum.at[dst_v.at[b]], add=True)
        return carry

    lax.fori_loop(0, NB, step, 0)
    plsc.subcore_barrier()
    pltpu.sync_copy(accum.at[pl.ds(s * RPT, RPT)], row_buf)
    pltpu.sync_copy(row_buf, out_hbm.at[c, pl.ds(s * RPT, RPT)])


def _tc_matmul(x_pad, W1, W2):
    def body(x_ref, w1_ref, w2_ref, y_ref):
        w = lax.dot(w1_ref[...], w2_ref[...],
                    preferred_element_type=jnp.float32)
        y_ref[...] = lax.dot(x_ref[...], w,
                             preferred_element_type=jnp.float32)
    return pl.pallas_call(
        body, out_shape=jax.ShapeDtypeStruct((NPAD, F), jnp.float32),
    )(x_pad, W1, W2)


def _tc_scale0(degP, Y):
    def body(degp_ref, y_ref, dis_ref, u0_ref):
        deg = degp_ref[0] + degp_ref[1] + 1.0
        dis = lax.rsqrt(deg)
        dis_ref[...] = dis
        u0_ref[...] = y_ref[...] * dis
    return pl.pallas_call(
        body,
        out_shape=(jax.ShapeDtypeStruct((NPAD, 1), jnp.float32),
                   jax.ShapeDtypeStruct((NPAD, F), jnp.float32)),
    )(degP, Y)


def _tc_combine(dis, P, u, power):
    def body(dis_ref, p_ref, u_ref, o_ref):
        d = dis_ref[...]
        f = d * d if power == 2 else d
        o_ref[...] = f * (p_ref[0] + p_ref[1] - u_ref[...])
    return pl.pallas_call(
        body, out_shape=jax.ShapeDtypeStruct((NPAD, F), jnp.float32),
    )(dis, P, u)


def kernel(x, edge_index, W1, W2):
    src = edge_index[0].astype(jnp.int32)
    dst = edge_index[1].astype(jnp.int32)
    srcg = jnp.full((EPAD,), N, jnp.int32).at[:E].set(src).reshape(NW, NB, BK)
    dstg = jnp.full((EPAD,), N, jnp.int32).at[:E].set(dst).reshape(NW, NB, BK)
    x_pad = jnp.zeros((NPAD, D), jnp.float32).at[:N].set(x)

    Y = _tc_matmul(x_pad, W1, W2)
    degP = _sc_degree(dstg)
    dis, u0 = _tc_scale0(degP.reshape(NC, NPAD, 1), Y)
    P = _sc_aggregate(u0, srcg, dstg)
    u1 = _tc_combine(dis, P, u0, power=2)
    Q = _sc_aggregate(u1, srcg, dstg)
    out = _tc_combine(dis, Q, u1, power=1)
    return out[:N]


# speedup vs baseline: 40.7021x; 1.1911x over previous
import functools

import jax
import jax.numpy as jnp
from jax import lax
from jax.experimental import pallas as pl
from jax.experimental.pallas import tpu as pltpu
from jax.experimental.pallas import tpu_sc as plsc

N = 10000
D = 128
F = 16
E = 320000
NC, NS = 2, 16
NW = NC * NS
BK = 128
NB = 80
KG = 8
NG = NB // KG
EPT = NB * BK
EPAD = NW * EPT
NPAD = 10240
RPT = NPAD // NS

_mesh = plsc.VectorSubcoreMesh(core_axis_name="c", subcore_axis_name="s")
_sc_params = pltpu.CompilerParams(use_tc_tiling_on_sc=False)


@functools.partial(
    pl.kernel,
    out_type=jax.ShapeDtypeStruct((NC, NPAD), jnp.float32),
    mesh=_mesh,
    compiler_params=_sc_params,
    scratch_types=[
        pltpu.VMEM_SHARED((NPAD,), jnp.float32),
        pltpu.VMEM((NB, BK), jnp.int32),
        pltpu.VMEM((BK,), jnp.float32),
        pltpu.VMEM((RPT,), jnp.float32),
    ],
)
def _sc_degree(dstg_hbm, out_hbm, accum, dst_v, ones_v, buf_v):
    c = lax.axis_index("c")
    s = lax.axis_index("s")
    tid = c * NS + s
    z16 = jnp.zeros((16,), jnp.float32)
    for j in range(RPT // 16):
        buf_v[pl.ds(j * 16, 16)] = z16
    o16 = jnp.full((16,), 1.0, jnp.float32)
    for j in range(BK // 16):
        ones_v[pl.ds(j * 16, 16)] = o16
    pltpu.sync_copy(buf_v, accum.at[pl.ds(s * RPT, RPT)])
    pltpu.sync_copy(dstg_hbm.at[tid], dst_v)
    plsc.subcore_barrier()

    def step(b, carry):
        pltpu.sync_copy(ones_v, accum.at[dst_v.at[b]], add=True)
        return carry

    lax.fori_loop(0, NB, step, 0)
    plsc.subcore_barrier()
    pltpu.sync_copy(accum.at[pl.ds(s * RPT, RPT)], buf_v)
    pltpu.sync_copy(buf_v, out_hbm.at[c, pl.ds(s * RPT, RPT)])


@functools.partial(
    pl.kernel,
    out_type=jax.ShapeDtypeStruct((NC, NPAD, F), jnp.float32),
    mesh=_mesh,
    compiler_params=_sc_params,
    scratch_types=[
        pltpu.VMEM_SHARED((NPAD, F), jnp.float32),
        pltpu.VMEM((RPT, F), jnp.float32),
        pltpu.VMEM((2, KG, BK, F), jnp.float32),
        pltpu.VMEM((NB, BK), jnp.int32),
        pltpu.VMEM((NB, BK), jnp.int32),
        pltpu.SemaphoreType.DMA((2,)),
        pltpu.SemaphoreType.DMA((2,)),
    ],
)
def _sc_aggregate(u_hbm, srcg_hbm, dstg_hbm, out_hbm,
                  accum, row_buf, buf, src_v, dst_v, gsem, ssem):
    c = lax.axis_index("c")
    s = lax.axis_index("s")
    tid = c * NS + s
    pltpu.sync_copy(u_hbm.at[pl.ds(s * RPT, RPT)], row_buf)
    pltpu.sync_copy(row_buf, accum.at[pl.ds(s * RPT, RPT)])
    pltpu.sync_copy(srcg_hbm.at[tid], src_v)
    pltpu.sync_copy(dstg_hbm.at[tid], dst_v)
    plsc.subcore_barrier()

    for j in range(KG):
        pltpu.async_copy(u_hbm.at[src_v.at[j]], buf.at[0, j], gsem.at[0])

    def body(g, carry):
        cur = g % 2
        nxt = (g + 1) % 2

        @pl.when(g >= 1)
        def _():
            for j in range(KG):
                pltpu.make_async_copy(
                    buf.at[nxt, j],
                    accum.at[dst_v.at[(g - 1) * KG + j]],
                    ssem.at[nxt]).wait()

        @pl.when(g + 1 < NG)
        def _():
            for j in range(KG):
                pltpu.async_copy(u_hbm.at[src_v.at[(g + 1) * KG + j]],
                                 buf.at[nxt, j], gsem.at[nxt])

        for j in range(KG):
            pltpu.make_async_copy(u_hbm.at[src_v.at[g * KG + j]],
                                  buf.at[cur, j], gsem.at[cur]).wait()
        for j in range(KG):
            pltpu.async_copy(buf.at[cur, j], accum.at[dst_v.at[g * KG + j]],
                             ssem.at[cur], add=True)
        return carry

    lax.fori_loop(0, NG, body, 0)
    last = (NG - 1) % 2
    for j in range(KG):
        pltpu.make_async_copy(buf.at[last, j],
                              accum.at[dst_v.at[(NG - 1) * KG + j]],
                              ssem.at[last]).wait()
    plsc.subcore_barrier()
    pltpu.sync_copy(accum.at[pl.ds(s * RPT, RPT)], row_buf)
    pltpu.sync_copy(row_buf, out_hbm.at[c, pl.ds(s * RPT, RPT)])


def _tc_matmul(x_pad, W1, W2):
    def body(x_ref, w1_ref, w2_ref, y_ref):
        w = lax.dot(w1_ref[...], w2_ref[...],
                    preferred_element_type=jnp.float32)
        y_ref[...] = lax.dot(x_ref[...], w,
                             preferred_element_type=jnp.float32)
    return pl.pallas_call(
        body, out_shape=jax.ShapeDtypeStruct((NPAD, F), jnp.float32),
    )(x_pad, W1, W2)


def _tc_scale0(degP, Y):
    def body(degp_ref, y_ref, dis_ref, u0_ref):
        deg = degp_ref[0] + degp_ref[1] + 1.0
        dis = lax.rsqrt(deg)
        dis_ref[...] = dis
        u0_ref[...] = y_ref[...] * dis
    return pl.pallas_call(
        body,
        out_shape=(jax.ShapeDtypeStruct((NPAD, 1), jnp.float32),
                   jax.ShapeDtypeStruct((NPAD, F), jnp.float32)),
    )(degP, Y)


def _tc_combine(dis, P, u, power):
    def body(dis_ref, p_ref, u_ref, o_ref):
        d = dis_ref[...]
        f = d * d if power == 2 else d
        o_ref[...] = f * (p_ref[0] + p_ref[1] - u_ref[...])
    return pl.pallas_call(
        body, out_shape=jax.ShapeDtypeStruct((NPAD, F), jnp.float32),
    )(dis, P, u)


def kernel(x, edge_index, W1, W2):
    src = edge_index[0].astype(jnp.int32)
    dst = edge_index[1].astype(jnp.int32)
    srcg = jnp.full((EPAD,), N, jnp.int32).at[:E].set(src).reshape(NW, NB, BK)
    dstg = jnp.full((EPAD,), N, jnp.int32).at[:E].set(dst).reshape(NW, NB, BK)
    x_pad = jnp.zeros((NPAD, D), jnp.float32).at[:N].set(x)

    Y = _tc_matmul(x_pad, W1, W2)
    degP = _sc_degree(dstg)
    dis, u0 = _tc_scale0(degP.reshape(NC, NPAD, 1), Y)
    P = _sc_aggregate(u0, srcg, dstg)
    u1 = _tc_combine(dis, P, u0, power=2)
    Q = _sc_aggregate(u1, srcg, dstg)
    out = _tc_combine(dis, Q, u1, power=1)
    return out[:N]


# speedup vs baseline: 55.3745x; 1.3605x over previous
import functools

import jax
import jax.numpy as jnp
from jax import lax
from jax.experimental import pallas as pl
from jax.experimental.pallas import tpu as pltpu
from jax.experimental.pallas import tpu_sc as plsc

N = 10000
D = 128
F = 16
E = 320000
NC, NS = 2, 16
NW = NC * NS
BK = 128
NB = 80
KG = 8
NG = NB // KG
EPT = NB * BK
EPAD = NW * EPT
NPAD = 10240
RPT = NPAD // NS

_mesh = plsc.VectorSubcoreMesh(core_axis_name="c", subcore_axis_name="s")
_sc_params = pltpu.CompilerParams(use_tc_tiling_on_sc=False)


@functools.partial(
    pl.kernel,
    out_type=jax.ShapeDtypeStruct((NC, NPAD), jnp.float32),
    mesh=_mesh,
    compiler_params=_sc_params,
    scratch_types=[
        pltpu.VMEM_SHARED((NPAD,), jnp.float32),
        pltpu.VMEM((NB, BK), jnp.int32),
        pltpu.VMEM((BK,), jnp.float32),
        pltpu.VMEM((RPT,), jnp.float32),
    ],
)
def _sc_degree(dstg_hbm, out_hbm, accum, dst_v, ones_v, buf_v):
    c = lax.axis_index("c")
    s = lax.axis_index("s")
    tid = c * NS + s
    z16 = jnp.zeros((16,), jnp.float32)
    for j in range(RPT // 16):
        buf_v[pl.ds(j * 16, 16)] = z16
    o16 = jnp.full((16,), 1.0, jnp.float32)
    for j in range(BK // 16):
        ones_v[pl.ds(j * 16, 16)] = o16
    pltpu.sync_copy(buf_v, accum.at[pl.ds(s * RPT, RPT)])
    pltpu.sync_copy(dstg_hbm.at[tid], dst_v)
    plsc.subcore_barrier()

    def step(b, carry):
        pltpu.sync_copy(ones_v, accum.at[dst_v.at[b]], add=True)
        return carry

    lax.fori_loop(0, NB, step, 0)
    plsc.subcore_barrier()
    pltpu.sync_copy(accum.at[pl.ds(s * RPT, RPT)], buf_v)
    pltpu.sync_copy(buf_v, out_hbm.at[c, pl.ds(s * RPT, RPT)])


@functools.partial(
    pl.kernel,
    out_type=jax.ShapeDtypeStruct((NC, NPAD, F), jnp.float32),
    mesh=_mesh,
    compiler_params=_sc_params,
    scratch_types=[
        pltpu.VMEM_SHARED((NPAD, F), jnp.float32),
        pltpu.VMEM_SHARED((NPAD, F), jnp.float32),
        pltpu.VMEM((RPT, F), jnp.float32),
        pltpu.VMEM((2, KG, BK, F), jnp.float32),
        pltpu.VMEM((NB, BK), jnp.int32),
        pltpu.VMEM((NB, BK), jnp.int32),
        pltpu.SemaphoreType.DMA((2,)),
        pltpu.SemaphoreType.DMA((2,)),
    ],
)
def _sc_aggregate(u_hbm, srcg_hbm, dstg_hbm, out_hbm,
                  accum, u_sp, row_buf, buf, src_v, dst_v, gsem, ssem):
    c = lax.axis_index("c")
    s = lax.axis_index("s")
    tid = c * NS + s
    pltpu.sync_copy(u_hbm.at[pl.ds(s * RPT, RPT)], row_buf)
    pltpu.sync_copy(row_buf, accum.at[pl.ds(s * RPT, RPT)])
    pltpu.sync_copy(row_buf, u_sp.at[pl.ds(s * RPT, RPT)])
    pltpu.sync_copy(srcg_hbm.at[tid], src_v)
    pltpu.sync_copy(dstg_hbm.at[tid], dst_v)
    plsc.subcore_barrier()

    for j in range(KG):
        pltpu.async_copy(u_sp.at[src_v.at[j]], buf.at[0, j], gsem.at[0])

    def body(g, carry):
        cur = g % 2
        nxt = (g + 1) % 2

        @pl.when(g >= 1)
        def _():
            for j in range(KG):
                pltpu.make_async_copy(
                    buf.at[nxt, j],
                    accum.at[dst_v.at[(g - 1) * KG + j]],
                    ssem.at[nxt]).wait()

        @pl.when(g + 1 < NG)
        def _():
            for j in range(KG):
                pltpu.async_copy(u_sp.at[src_v.at[(g + 1) * KG + j]],
                                 buf.at[nxt, j], gsem.at[nxt])

        for j in range(KG):
            pltpu.make_async_copy(u_sp.at[src_v.at[g * KG + j]],
                                  buf.at[cur, j], gsem.at[cur]).wait()
        for j in range(KG):
            pltpu.async_copy(buf.at[cur, j], accum.at[dst_v.at[g * KG + j]],
                             ssem.at[cur], add=True)
        return carry

    lax.fori_loop(0, NG, body, 0)
    last = (NG - 1) % 2
    for j in range(KG):
        pltpu.make_async_copy(buf.at[last, j],
                              accum.at[dst_v.at[(NG - 1) * KG + j]],
                              ssem.at[last]).wait()
    plsc.subcore_barrier()
    pltpu.sync_copy(accum.at[pl.ds(s * RPT, RPT)], row_buf)
    pltpu.sync_copy(row_buf, out_hbm.at[c, pl.ds(s * RPT, RPT)])


def _tc_matmul(x_pad, W1, W2):
    def body(x_ref, w1_ref, w2_ref, y_ref):
        w = lax.dot(w1_ref[...], w2_ref[...],
                    preferred_element_type=jnp.float32)
        y_ref[...] = lax.dot(x_ref[...], w,
                             preferred_element_type=jnp.float32)
    return pl.pallas_call(
        body, out_shape=jax.ShapeDtypeStruct((NPAD, F), jnp.float32),
    )(x_pad, W1, W2)


def _tc_scale0(degP, Y):
    def body(degp_ref, y_ref, dis_ref, u0_ref):
        deg = degp_ref[0] + degp_ref[1] + 1.0
        dis = lax.rsqrt(deg)
        dis_ref[...] = dis
        u0_ref[...] = y_ref[...] * dis
    return pl.pallas_call(
        body,
        out_shape=(jax.ShapeDtypeStruct((NPAD, 1), jnp.float32),
                   jax.ShapeDtypeStruct((NPAD, F), jnp.float32)),
    )(degP, Y)


def _tc_combine(dis, P, u, power):
    def body(dis_ref, p_ref, u_ref, o_ref):
        d = dis_ref[...]
        f = d * d if power == 2 else d
        o_ref[...] = f * (p_ref[0] + p_ref[1] - u_ref[...])
    return pl.pallas_call(
        body, out_shape=jax.ShapeDtypeStruct((NPAD, F), jnp.float32),
    )(dis, P, u)


def kernel(x, edge_index, W1, W2):
    src = edge_index[0].astype(jnp.int32)
    dst = edge_index[1].astype(jnp.int32)
    srcg = jnp.full((EPAD,), N, jnp.int32).at[:E].set(src).reshape(NW, NB, BK)
    dstg = jnp.full((EPAD,), N, jnp.int32).at[:E].set(dst).reshape(NW, NB, BK)
    x_pad = jnp.zeros((NPAD, D), jnp.float32).at[:N].set(x)

    Y = _tc_matmul(x_pad, W1, W2)
    degP = _sc_degree(dstg)
    dis, u0 = _tc_scale0(degP.reshape(NC, NPAD, 1), Y)
    P = _sc_aggregate(u0, srcg, dstg)
    u1 = _tc_combine(dis, P, u0, power=2)
    Q = _sc_aggregate(u1, srcg, dstg)
    out = _tc_combine(dis, Q, u1, power=1)
    return out[:N]


# speedup vs baseline: 59.0589x; 1.0665x over previous
import functools

import jax
import jax.numpy as jnp
from jax import lax
from jax.experimental import pallas as pl
from jax.experimental.pallas import tpu as pltpu
from jax.experimental.pallas import tpu_sc as plsc

N = 10000
D = 128
F = 16
E = 320000
NC, NS = 2, 16
NW = NC * NS
BK = 128
NB = 80
KG = 8
NG = NB // KG
EPT = NB * BK
EPAD = NW * EPT
NPAD = 10240
RPT = NPAD // NS

_mesh = plsc.VectorSubcoreMesh(core_axis_name="c", subcore_axis_name="s")
_sc_params = pltpu.CompilerParams(use_tc_tiling_on_sc=False)


@functools.partial(
    pl.kernel,
    out_type=jax.ShapeDtypeStruct((NC, NPAD), jnp.float32),
    mesh=_mesh,
    compiler_params=_sc_params,
    scratch_types=[
        pltpu.VMEM_SHARED((NPAD,), jnp.float32),
        pltpu.VMEM((NB, BK), jnp.int32),
        pltpu.VMEM((BK,), jnp.float32),
        pltpu.VMEM((RPT,), jnp.float32),
    ],
)
def _sc_degree(dstg_hbm, out_hbm, accum, dst_v, ones_v, buf_v):
    c = lax.axis_index("c")
    s = lax.axis_index("s")
    tid = c * NS + s
    z16 = jnp.zeros((16,), jnp.float32)
    for j in range(RPT // 16):
        buf_v[pl.ds(j * 16, 16)] = z16
    o16 = jnp.full((16,), 1.0, jnp.float32)
    for j in range(BK // 16):
        ones_v[pl.ds(j * 16, 16)] = o16
    pltpu.sync_copy(buf_v, accum.at[pl.ds(s * RPT, RPT)])
    pltpu.sync_copy(dstg_hbm.at[tid], dst_v)
    plsc.subcore_barrier()

    def step(b, carry):
        pltpu.sync_copy(ones_v, accum.at[dst_v.at[b]], add=True)
        return carry

    lax.fori_loop(0, NB, step, 0)
    plsc.subcore_barrier()
    pltpu.sync_copy(accum.at[pl.ds(s * RPT, RPT)], buf_v)
    pltpu.sync_copy(buf_v, out_hbm.at[c, pl.ds(s * RPT, RPT)])


_AGG_SCRATCH = [
    pltpu.VMEM_SHARED((NPAD, F), jnp.float32),
    pltpu.VMEM_SHARED((NPAD, F), jnp.float32),
    pltpu.VMEM((RPT, F), jnp.float32),
    pltpu.VMEM((2, KG, BK, F), jnp.float32),
    pltpu.VMEM((NB, BK), jnp.int32),
    pltpu.VMEM((NB, BK), jnp.int32),
    pltpu.SemaphoreType.DMA((2,)),
    pltpu.SemaphoreType.DMA((2,)),
]


def _edge_pipeline(u_sp, accum, buf, src_v, dst_v, gsem, ssem):
    for j in range(KG):
        pltpu.async_copy(u_sp.at[src_v.at[j]], buf.at[0, j], gsem.at[0])

    def body(g, carry):
        cur = g % 2
        nxt = (g + 1) % 2

        @pl.when(g >= 1)
        def _():
            for j in range(KG):
                pltpu.make_async_copy(
                    buf.at[nxt, j],
                    accum.at[dst_v.at[(g - 1) * KG + j]],
                    ssem.at[nxt]).wait()

        @pl.when(g + 1 < NG)
        def _():
            for j in range(KG):
                pltpu.async_copy(u_sp.at[src_v.at[(g + 1) * KG + j]],
                                 buf.at[nxt, j], gsem.at[nxt])

        for j in range(KG):
            pltpu.make_async_copy(u_sp.at[src_v.at[g * KG + j]],
                                  buf.at[cur, j], gsem.at[cur]).wait()
        for j in range(KG):
            pltpu.async_copy(buf.at[cur, j], accum.at[dst_v.at[g * KG + j]],
                             ssem.at[cur], add=True)
        return carry

    lax.fori_loop(0, NG, body, 0)
    last = (NG - 1) % 2
    for j in range(KG):
        pltpu.make_async_copy(buf.at[last, j],
                              accum.at[dst_v.at[(NG - 1) * KG + j]],
                              ssem.at[last]).wait()


@functools.partial(
    pl.kernel,
    out_type=jax.ShapeDtypeStruct((NC, NPAD, F), jnp.float32),
    mesh=_mesh,
    compiler_params=_sc_params,
    scratch_types=_AGG_SCRATCH,
)
def _sc_agg_plain(u_hbm, srcg_hbm, dstg_hbm, out_hbm,
                  accum, u_sp, row_buf, buf, src_v, dst_v, gsem, ssem):
    c = lax.axis_index("c")
    s = lax.axis_index("s")
    tid = c * NS + s
    pltpu.sync_copy(u_hbm.at[pl.ds(s * RPT, RPT)], row_buf)
    pltpu.sync_copy(row_buf, accum.at[pl.ds(s * RPT, RPT)])
    pltpu.sync_copy(row_buf, u_sp.at[pl.ds(s * RPT, RPT)])
    pltpu.sync_copy(srcg_hbm.at[tid], src_v)
    pltpu.sync_copy(dstg_hbm.at[tid], dst_v)
    plsc.subcore_barrier()
    _edge_pipeline(u_sp, accum, buf, src_v, dst_v, gsem, ssem)
    plsc.subcore_barrier()
    pltpu.sync_copy(accum.at[pl.ds(s * RPT, RPT)], row_buf)
    pltpu.sync_copy(row_buf, out_hbm.at[c, pl.ds(s * RPT, RPT)])


@functools.partial(
    pl.kernel,
    out_type=(jax.ShapeDtypeStruct((NC, NPAD, F), jnp.float32),
              jax.ShapeDtypeStruct((NPAD, F), jnp.float32)),
    mesh=_mesh,
    compiler_params=_sc_params,
    scratch_types=_AGG_SCRATCH + [pltpu.VMEM((RPT, F), jnp.float32)] * 3,
)
def _sc_agg_fused(p_hbm, u0_hbm, d2_hbm, srcg_hbm, dstg_hbm,
                  out_hbm, u1_hbm,
                  accum, u_sp, row_buf, buf, src_v, dst_v, gsem, ssem,
                  pa_buf, pb_buf, d2_buf):
    c = lax.axis_index("c")
    s = lax.axis_index("s")
    tid = c * NS + s
    pltpu.sync_copy(p_hbm.at[0, pl.ds(s * RPT, RPT)], pa_buf)
    pltpu.sync_copy(p_hbm.at[1, pl.ds(s * RPT, RPT)], pb_buf)
    pltpu.sync_copy(u0_hbm.at[pl.ds(s * RPT, RPT)], row_buf)
    pltpu.sync_copy(d2_hbm.at[pl.ds(s * RPT, RPT)], d2_buf)
    pltpu.sync_copy(srcg_hbm.at[tid], src_v)
    pltpu.sync_copy(dstg_hbm.at[tid], dst_v)

    def rows16(r, carry):
        base = r * 16
        for j in range(16):
            row = base + j
            row_buf[row] = d2_buf[row] * (pa_buf[row] + pb_buf[row]
                                          - row_buf[row])
        return carry

    lax.fori_loop(0, RPT // 16, rows16, 0)
    pltpu.sync_copy(row_buf, accum.at[pl.ds(s * RPT, RPT)])
    pltpu.sync_copy(row_buf, u_sp.at[pl.ds(s * RPT, RPT)])

    @pl.when(c == 0)
    def _():
        pltpu.sync_copy(row_buf, u1_hbm.at[pl.ds(s * RPT, RPT)])

    plsc.subcore_barrier()
    _edge_pipeline(u_sp, accum, buf, src_v, dst_v, gsem, ssem)
    plsc.subcore_barrier()
    pltpu.sync_copy(accum.at[pl.ds(s * RPT, RPT)], row_buf)
    pltpu.sync_copy(row_buf, out_hbm.at[c, pl.ds(s * RPT, RPT)])


def _tc_matmul(x_pad, W1, W2):
    def body(x_ref, w1_ref, w2_ref, y_ref):
        w = lax.dot(w1_ref[...], w2_ref[...],
                    preferred_element_type=jnp.float32)
        y_ref[...] = lax.dot(x_ref[...], w,
                             preferred_element_type=jnp.float32)
    return pl.pallas_call(
        body, out_shape=jax.ShapeDtypeStruct((NPAD, F), jnp.float32),
    )(x_pad, W1, W2)


def _tc_scale0(degP, Y):
    def body(degp_ref, y_ref, disr_ref, dis2r_ref, u0_ref):
        deg = degp_ref[0] + degp_ref[1] + 1.0
        dis = lax.rsqrt(deg)
        ones = jnp.ones((1, F), jnp.float32)
        disr_ref[...] = dis * ones
        dis2r_ref[...] = (dis * dis) * ones
        u0_ref[...] = y_ref[...] * dis
    return pl.pallas_call(
        body,
        out_shape=(jax.ShapeDtypeStruct((NPAD, F), jnp.float32),
                   jax.ShapeDtypeStruct((NPAD, F), jnp.float32),
                   jax.ShapeDtypeStruct((NPAD, F), jnp.float32)),
    )(degP, Y)


def _tc_final(disr, Q, u1):
    def body(disr_ref, q_ref, u_ref, o_ref):
        o_ref[...] = disr_ref[...] * (q_ref[0] + q_ref[1] - u_ref[...])
    return pl.pallas_call(
        body, out_shape=jax.ShapeDtypeStruct((NPAD, F), jnp.float32),
    )(disr, Q, u1)


def kernel(x, edge_index, W1, W2):
    src = edge_index[0].astype(jnp.int32)
    dst = edge_index[1].astype(jnp.int32)
    srcg = jnp.full((EPAD,), N, jnp.int32).at[:E].set(src).reshape(NW, NB, BK)
    dstg = jnp.full((EPAD,), N, jnp.int32).at[:E].set(dst).reshape(NW, NB, BK)
    x_pad = jnp.zeros((NPAD, D), jnp.float32).at[:N].set(x)

    Y = _tc_matmul(x_pad, W1, W2)
    degP = _sc_degree(dstg)
    disr, dis2r, u0 = _tc_scale0(degP.reshape(NC, NPAD, 1), Y)
    P = _sc_agg_plain(u0, srcg, dstg)
    Q, u1 = _sc_agg_fused(P, u0, dis2r, srcg, dstg)
    out = _tc_final(disr, Q, u1)
    return out[:N]


# speedup vs baseline: 66.2003x; 1.1209x over previous
import functools

import jax
import jax.numpy as jnp
from jax import lax
from jax.experimental import pallas as pl
from jax.experimental.pallas import tpu as pltpu
from jax.experimental.pallas import tpu_sc as plsc

N = 10000
D = 128
F = 16
E = 320000
NC, NS = 2, 16
NW = NC * NS
BK = 128
NB = 80
KG = 8
NG = NB // KG
EPT = NB * BK
EPAD = NW * EPT
NPAD = 10240
RPT = NPAD // NS

_mesh = plsc.VectorSubcoreMesh(core_axis_name="c", subcore_axis_name="s")
_sc_params = pltpu.CompilerParams(use_tc_tiling_on_sc=False)


@functools.partial(
    pl.kernel,
    out_type=jax.ShapeDtypeStruct((NC, NPAD), jnp.float32),
    mesh=_mesh,
    compiler_params=_sc_params,
    scratch_types=[
        pltpu.VMEM_SHARED((NPAD,), jnp.float32),
        pltpu.VMEM((NB, BK), jnp.int32),
        pltpu.VMEM((BK,), jnp.float32),
        pltpu.VMEM((RPT,), jnp.float32),
    ],
)
def _sc_degree(dstg_hbm, out_hbm, accum, dst_v, ones_v, buf_v):
    c = lax.axis_index("c")
    s = lax.axis_index("s")
    tid = c * NS + s
    z16 = jnp.zeros((16,), jnp.float32)
    for j in range(RPT // 16):
        buf_v[pl.ds(j * 16, 16)] = z16
    o16 = jnp.full((16,), 1.0, jnp.float32)
    for j in range(BK // 16):
        ones_v[pl.ds(j * 16, 16)] = o16
    pltpu.sync_copy(buf_v, accum.at[pl.ds(s * RPT, RPT)])
    pltpu.sync_copy(dstg_hbm.at[tid], dst_v)
    plsc.subcore_barrier()

    def step(b, carry):
        pltpu.sync_copy(ones_v, accum.at[dst_v.at[b]], add=True)
        return carry

    lax.fori_loop(0, NB, step, 0)
    plsc.subcore_barrier()
    pltpu.sync_copy(accum.at[pl.ds(s * RPT, RPT)], buf_v)
    pltpu.sync_copy(buf_v, out_hbm.at[c, pl.ds(s * RPT, RPT)])


_AGG_SCRATCH = [
    pltpu.VMEM_SHARED((NPAD, F), jnp.float32),
    pltpu.VMEM_SHARED((NPAD, F), jnp.float32),
    pltpu.VMEM((RPT, F), jnp.float32),
    pltpu.VMEM((2, KG, BK, F), jnp.float32),
    pltpu.VMEM((NB, BK), jnp.int32),
    pltpu.VMEM((NB, BK), jnp.int32),
    pltpu.SemaphoreType.DMA((2,)),
    pltpu.SemaphoreType.DMA((2,)),
]


def _edge_pipeline(u_sp, accum, buf, src_v, dst_v, gsem, ssem):
    for j in range(KG):
        pltpu.async_copy(u_sp.at[src_v.at[j]], buf.at[0, j], gsem.at[0])

    def body(g, carry):
        cur = g % 2
        nxt = (g + 1) % 2

        @pl.when(g >= 1)
        def _():
            for j in range(KG):
                pltpu.make_async_copy(
                    buf.at[nxt, j],
                    accum.at[dst_v.at[(g - 1) * KG + j]],
                    ssem.at[nxt]).wait()

        @pl.when(g + 1 < NG)
        def _():
            for j in range(KG):
                pltpu.async_copy(u_sp.at[src_v.at[(g + 1) * KG + j]],
                                 buf.at[nxt, j], gsem.at[nxt])

        for j in range(KG):
            pltpu.make_async_copy(u_sp.at[src_v.at[g * KG + j]],
                                  buf.at[cur, j], gsem.at[cur]).wait()
        for j in range(KG):
            pltpu.async_copy(buf.at[cur, j], accum.at[dst_v.at[g * KG + j]],
                             ssem.at[cur], add=True)
        return carry

    lax.fori_loop(0, NG, body, 0)
    last = (NG - 1) % 2
    for j in range(KG):
        pltpu.make_async_copy(buf.at[last, j],
                              accum.at[dst_v.at[(NG - 1) * KG + j]],
                              ssem.at[last]).wait()


@functools.partial(
    pl.kernel,
    out_type=(jax.ShapeDtypeStruct((NC, NPAD, F), jnp.float32),
              jax.ShapeDtypeStruct((NPAD, F), jnp.float32)),
    mesh=_mesh,
    compiler_params=_sc_params,
    scratch_types=_AGG_SCRATCH + [
        pltpu.VMEM((RPT, F), jnp.float32),
        pltpu.VMEM((RPT, F), jnp.float32),
    ],
)
def _sc_agg_first(disr_hbm, y_hbm, srcg_hbm, dstg_hbm,
                  out_hbm, u0_hbm,
                  accum, u_sp, row_buf, buf, src_v, dst_v, gsem, ssem,
                  y_buf, disr_buf):
    c = lax.axis_index("c")
    s = lax.axis_index("s")
    tid = c * NS + s
    pltpu.sync_copy(disr_hbm.at[pl.ds(s * RPT, RPT)], disr_buf)
    pltpu.sync_copy(y_hbm.at[pl.ds(s * RPT, RPT)], y_buf)
    pltpu.sync_copy(srcg_hbm.at[tid], src_v)
    pltpu.sync_copy(dstg_hbm.at[tid], dst_v)

    def rows16(r, carry):
        base = r * 16
        for j in range(16):
            row = base + j
            row_buf[row] = disr_buf[row] * y_buf[row]
        return carry

    lax.fori_loop(0, RPT // 16, rows16, 0)
    pltpu.sync_copy(row_buf, accum.at[pl.ds(s * RPT, RPT)])
    pltpu.sync_copy(row_buf, u_sp.at[pl.ds(s * RPT, RPT)])

    @pl.when(c == 0)
    def _():
        pltpu.sync_copy(row_buf, u0_hbm.at[pl.ds(s * RPT, RPT)])

    plsc.subcore_barrier()
    _edge_pipeline(u_sp, accum, buf, src_v, dst_v, gsem, ssem)
    plsc.subcore_barrier()
    pltpu.sync_copy(accum.at[pl.ds(s * RPT, RPT)], row_buf)
    pltpu.sync_copy(row_buf, out_hbm.at[c, pl.ds(s * RPT, RPT)])


@functools.partial(
    pl.kernel,
    out_type=(jax.ShapeDtypeStruct((NC, NPAD, F), jnp.float32),
              jax.ShapeDtypeStruct((NPAD, F), jnp.float32)),
    mesh=_mesh,
    compiler_params=_sc_params,
    scratch_types=_AGG_SCRATCH + [pltpu.VMEM((RPT, F), jnp.float32)] * 3,
)
def _sc_agg_fused(p_hbm, u0_hbm, disr_hbm, srcg_hbm, dstg_hbm,
                  out_hbm, u1_hbm,
                  accum, u_sp, row_buf, buf, src_v, dst_v, gsem, ssem,
                  pa_buf, pb_buf, d_buf):
    c = lax.axis_index("c")
    s = lax.axis_index("s")
    tid = c * NS + s
    pltpu.sync_copy(p_hbm.at[0, pl.ds(s * RPT, RPT)], pa_buf)
    pltpu.sync_copy(p_hbm.at[1, pl.ds(s * RPT, RPT)], pb_buf)
    pltpu.sync_copy(u0_hbm.at[pl.ds(s * RPT, RPT)], row_buf)
    pltpu.sync_copy(disr_hbm.at[pl.ds(s * RPT, RPT)], d_buf)
    pltpu.sync_copy(srcg_hbm.at[tid], src_v)
    pltpu.sync_copy(dstg_hbm.at[tid], dst_v)

    def rows16(r, carry):
        base = r * 16
        for j in range(16):
            row = base + j
            d = d_buf[row]
            row_buf[row] = d * d * (pa_buf[row] + pb_buf[row]
                                    - row_buf[row])
        return carry

    lax.fori_loop(0, RPT // 16, rows16, 0)
    pltpu.sync_copy(row_buf, accum.at[pl.ds(s * RPT, RPT)])
    pltpu.sync_copy(row_buf, u_sp.at[pl.ds(s * RPT, RPT)])

    @pl.when(c == 0)
    def _():
        pltpu.sync_copy(row_buf, u1_hbm.at[pl.ds(s * RPT, RPT)])

    plsc.subcore_barrier()
    _edge_pipeline(u_sp, accum, buf, src_v, dst_v, gsem, ssem)
    plsc.subcore_barrier()
    pltpu.sync_copy(accum.at[pl.ds(s * RPT, RPT)], row_buf)
    pltpu.sync_copy(row_buf, out_hbm.at[c, pl.ds(s * RPT, RPT)])


def _tc_matmul(x_pad, W1, W2):
    def body(x_ref, w1_ref, w2_ref, y_ref):
        w = lax.dot(w1_ref[...], w2_ref[...],
                    preferred_element_type=jnp.float32)
        y_ref[...] = lax.dot(x_ref[...], w,
                             preferred_element_type=jnp.float32)
    return pl.pallas_call(
        body, out_shape=jax.ShapeDtypeStruct((NPAD, F), jnp.float32),
    )(x_pad, W1, W2)


def _tc_rsqrt(degP):
    def body(deg_ref, dis_ref):
        dis_ref[...] = lax.rsqrt(deg_ref[0] + deg_ref[1] + 1.0)
    return pl.pallas_call(
        body, out_shape=jax.ShapeDtypeStruct((NPAD // BK, BK), jnp.float32),
    )(degP.reshape(NC, NPAD // BK, BK))


def _tc_final(disr, Q, u1):
    def body(disr_ref, q_ref, u_ref, o_ref):
        o_ref[...] = disr_ref[...] * (q_ref[0] + q_ref[1] - u_ref[...])
    return pl.pallas_call(
        body, out_shape=jax.ShapeDtypeStruct((NPAD, F), jnp.float32),
    )(disr, Q, u1)


def kernel(x, edge_index, W1, W2):
    src = edge_index[0].astype(jnp.int32)
    dst = edge_index[1].astype(jnp.int32)
    srcg = jnp.full((EPAD,), N, jnp.int32).at[:E].set(src).reshape(NW, NB, BK)
    dstg = jnp.full((EPAD,), N, jnp.int32).at[:E].set(dst).reshape(NW, NB, BK)
    x_pad = jnp.zeros((NPAD, D), jnp.float32).at[:N].set(x)

    Y = _tc_matmul(x_pad, W1, W2)
    degP = _sc_degree(dstg)
    dis = _tc_rsqrt(degP)
    disr = jnp.broadcast_to(dis.reshape(NPAD, 1), (NPAD, F))
    P, u0 = _sc_agg_first(disr, Y, srcg, dstg)
    Q, u1 = _sc_agg_fused(P, u0, disr, srcg, dstg)
    out = _tc_final(disr, Q, u1)
    return out[:N]


# speedup vs baseline: 68.5103x; 1.0349x over previous
import functools

import jax
import jax.numpy as jnp
from jax import lax
from jax.experimental import pallas as pl
from jax.experimental.pallas import tpu as pltpu
from jax.experimental.pallas import tpu_sc as plsc

N = 10000
D = 128
F = 16
E = 320000
NC, NS = 2, 16
NW = NC * NS
BK = 128
NB = 80
KG = 8
NG = NB // KG
EPT = NB * BK
EPAD = NW * EPT
NPAD = 10240
RPT = NPAD // NS

_mesh = plsc.VectorSubcoreMesh(core_axis_name="c", subcore_axis_name="s")
_sc_params = pltpu.CompilerParams(use_tc_tiling_on_sc=False)


@functools.partial(
    pl.kernel,
    out_type=jax.ShapeDtypeStruct((NC, NPAD), jnp.float32),
    mesh=_mesh,
    compiler_params=_sc_params,
    scratch_types=[
        pltpu.VMEM_SHARED((NPAD,), jnp.float32),
        pltpu.VMEM((NB, BK), jnp.int32),
        pltpu.VMEM((BK,), jnp.float32),
        pltpu.VMEM((RPT,), jnp.float32),
    ],
)
def _sc_degree(dstg_hbm, out_hbm, accum, dst_v, ones_v, buf_v):
    c = lax.axis_index("c")
    s = lax.axis_index("s")
    tid = c * NS + s
    z16 = jnp.zeros((16,), jnp.float32)
    for j in range(RPT // 16):
        buf_v[pl.ds(j * 16, 16)] = z16
    o16 = jnp.full((16,), 1.0, jnp.float32)
    for j in range(BK // 16):
        ones_v[pl.ds(j * 16, 16)] = o16
    pltpu.sync_copy(buf_v, accum.at[pl.ds(s * RPT, RPT)])
    pltpu.sync_copy(dstg_hbm.at[tid], dst_v)
    plsc.subcore_barrier()

    def step(b, carry):
        pltpu.sync_copy(ones_v, accum.at[dst_v.at[b]], add=True)
        return carry

    lax.fori_loop(0, NB, step, 0)
    plsc.subcore_barrier()
    pltpu.sync_copy(accum.at[pl.ds(s * RPT, RPT)], buf_v)
    pltpu.sync_copy(buf_v, out_hbm.at[c, pl.ds(s * RPT, RPT)])


_AGG_SCRATCH = [
    pltpu.VMEM_SHARED((NPAD, F), jnp.float32),
    pltpu.VMEM_SHARED((NPAD, F), jnp.float32),
    pltpu.VMEM((RPT, F), jnp.float32),
    pltpu.VMEM((2, KG, BK, F), jnp.float32),
    pltpu.VMEM((NB, BK), jnp.int32),
    pltpu.VMEM((NB, BK), jnp.int32),
    pltpu.SemaphoreType.DMA((2,)),
    pltpu.SemaphoreType.DMA((2,)),
]


def _edge_pipeline(u_sp, accum, buf, src_v, dst_v, gsem, ssem):
    for j in range(KG):
        pltpu.async_copy(u_sp.at[src_v.at[j]], buf.at[0, j], gsem.at[0])

    def body(g, carry):
        cur = g % 2
        nxt = (g + 1) % 2

        @pl.when(g >= 1)
        def _():
            for j in range(KG):
                pltpu.make_async_copy(
                    buf.at[nxt, j],
                    accum.at[dst_v.at[(g - 1) * KG + j]],
                    ssem.at[nxt]).wait()

        @pl.when(g + 1 < NG)
        def _():
            for j in range(KG):
                pltpu.async_copy(u_sp.at[src_v.at[(g + 1) * KG + j]],
                                 buf.at[nxt, j], gsem.at[nxt])

        for j in range(KG):
            pltpu.make_async_copy(u_sp.at[src_v.at[g * KG + j]],
                                  buf.at[cur, j], gsem.at[cur]).wait()
        for j in range(KG):
            pltpu.async_copy(buf.at[cur, j], accum.at[dst_v.at[g * KG + j]],
                             ssem.at[cur], add=True)
        return carry

    lax.fori_loop(0, NG, body, 0)
    last = (NG - 1) % 2
    for j in range(KG):
        pltpu.make_async_copy(buf.at[last, j],
                              accum.at[dst_v.at[(NG - 1) * KG + j]],
                              ssem.at[last]).wait()


@functools.partial(
    pl.kernel,
    out_type=(jax.ShapeDtypeStruct((NC, NPAD, F), jnp.float32),
              jax.ShapeDtypeStruct((NPAD, F), jnp.float32)),
    mesh=_mesh,
    compiler_params=_sc_params,
    scratch_types=_AGG_SCRATCH + [
        pltpu.VMEM((RPT, F), jnp.float32),
        pltpu.VMEM((RPT, F), jnp.float32),
    ],
)
def _sc_agg_first(disr_hbm, y_hbm, srcg_hbm, dstg_hbm,
                  out_hbm, u0_hbm,
                  accum, u_sp, row_buf, buf, src_v, dst_v, gsem, ssem,
                  y_buf, disr_buf):
    c = lax.axis_index("c")
    s = lax.axis_index("s")
    tid = c * NS + s
    pltpu.sync_copy(disr_hbm.at[pl.ds(s * RPT, RPT)], disr_buf)
    pltpu.sync_copy(y_hbm.at[pl.ds(s * RPT, RPT)], y_buf)
    pltpu.sync_copy(srcg_hbm.at[tid], src_v)
    pltpu.sync_copy(dstg_hbm.at[tid], dst_v)

    def rows16(r, carry):
        base = r * 16
        for j in range(16):
            row = base + j
            row_buf[row] = disr_buf[row] * y_buf[row]
        return carry

    lax.fori_loop(0, RPT // 16, rows16, 0)
    pltpu.sync_copy(row_buf, accum.at[pl.ds(s * RPT, RPT)])
    pltpu.sync_copy(row_buf, u_sp.at[pl.ds(s * RPT, RPT)])

    @pl.when(c == 0)
    def _():
        pltpu.sync_copy(row_buf, u0_hbm.at[pl.ds(s * RPT, RPT)])

    plsc.subcore_barrier()
    _edge_pipeline(u_sp, accum, buf, src_v, dst_v, gsem, ssem)
    plsc.subcore_barrier()
    pltpu.sync_copy(accum.at[pl.ds(s * RPT, RPT)], row_buf)
    pltpu.sync_copy(row_buf, out_hbm.at[c, pl.ds(s * RPT, RPT)])


@functools.partial(
    pl.kernel,
    out_type=(jax.ShapeDtypeStruct((NC, NPAD, F), jnp.float32),
              jax.ShapeDtypeStruct((NPAD, F), jnp.float32)),
    mesh=_mesh,
    compiler_params=_sc_params,
    scratch_types=_AGG_SCRATCH + [pltpu.VMEM((RPT, F), jnp.float32)] * 3,
)
def _sc_agg_fused(p_hbm, u0_hbm, disr_hbm, srcg_hbm, dstg_hbm,
                  out_hbm, u1_hbm,
                  accum, u_sp, row_buf, buf, src_v, dst_v, gsem, ssem,
                  pa_buf, pb_buf, d_buf):
    c = lax.axis_index("c")
    s = lax.axis_index("s")
    tid = c * NS + s
    pltpu.sync_copy(p_hbm.at[0, pl.ds(s * RPT, RPT)], pa_buf)
    pltpu.sync_copy(p_hbm.at[1, pl.ds(s * RPT, RPT)], pb_buf)
    pltpu.sync_copy(u0_hbm.at[pl.ds(s * RPT, RPT)], row_buf)
    pltpu.sync_copy(disr_hbm.at[pl.ds(s * RPT, RPT)], d_buf)
    pltpu.sync_copy(srcg_hbm.at[tid], src_v)
    pltpu.sync_copy(dstg_hbm.at[tid], dst_v)

    def rows16(r, carry):
        base = r * 16
        for j in range(16):
            row = base + j
            d = d_buf[row]
            row_buf[row] = d * d * (pa_buf[row] + pb_buf[row]
                                    - row_buf[row])
        return carry

    lax.fori_loop(0, RPT // 16, rows16, 0)
    pltpu.sync_copy(row_buf, accum.at[pl.ds(s * RPT, RPT)])
    pltpu.sync_copy(row_buf, u_sp.at[pl.ds(s * RPT, RPT)])

    @pl.when(c == 0)
    def _():
        pltpu.sync_copy(row_buf, u1_hbm.at[pl.ds(s * RPT, RPT)])

    plsc.subcore_barrier()
    _edge_pipeline(u_sp, accum, buf, src_v, dst_v, gsem, ssem)
    plsc.subcore_barrier()
    pltpu.sync_copy(accum.at[pl.ds(s * RPT, RPT)], row_buf)
    pltpu.sync_copy(row_buf, out_hbm.at[c, pl.ds(s * RPT, RPT)])


def _tc_matmul(x, W1, W2):
    def body(x_ref, w1_ref, w2_ref, y_ref):
        w = lax.dot(w1_ref[...], w2_ref[...],
                    preferred_element_type=jnp.float32)
        y_ref[pl.ds(0, N)] = lax.dot(x_ref[...], w,
                                     preferred_element_type=jnp.float32)
        y_ref[pl.ds(N, NPAD - N)] = jnp.zeros((NPAD - N, F), jnp.float32)
    return pl.pallas_call(
        body, out_shape=jax.ShapeDtypeStruct((NPAD, F), jnp.float32),
    )(x, W1, W2)


def _tc_rsqrt(degP):
    def body(deg_ref, dis_ref):
        dis_ref[...] = lax.rsqrt(deg_ref[0] + deg_ref[1] + 1.0)
    return pl.pallas_call(
        body, out_shape=jax.ShapeDtypeStruct((NPAD // BK, BK), jnp.float32),
    )(degP.reshape(NC, NPAD // BK, BK))


def _tc_final(disr, Q, u1):
    def body(disr_ref, q_ref, u_ref, o_ref):
        o_ref[...] = disr_ref[pl.ds(0, N)] * (
            q_ref[0, pl.ds(0, N)] + q_ref[1, pl.ds(0, N)]
            - u_ref[pl.ds(0, N)])
    return pl.pallas_call(
        body, out_shape=jax.ShapeDtypeStruct((N, F), jnp.float32),
    )(disr, Q, u1)


def kernel(x, edge_index, W1, W2):
    src = edge_index[0].astype(jnp.int32)
    dst = edge_index[1].astype(jnp.int32)
    srcg = jnp.full((EPAD,), N, jnp.int32).at[:E].set(src).reshape(NW, NB, BK)
    dstg = jnp.full((EPAD,), N, jnp.int32).at[:E].set(dst).reshape(NW, NB, BK)

    Y = _tc_matmul(x, W1, W2)
    degP = _sc_degree(dstg)
    dis = _tc_rsqrt(degP)
    disr = jnp.broadcast_to(dis.reshape(NPAD, 1), (NPAD, F))
    P, u0 = _sc_agg_first(disr, Y, srcg, dstg)
    Q, u1 = _sc_agg_fused(P, u0, disr, srcg, dstg)
    return _tc_final(disr, Q, u1)
